# baseline (speedup 1.0000x reference)
import jax
import jax.numpy as jnp
from jax.experimental import pallas as pl
from jax.experimental.pallas import tpu as pltpu

_BS = 32
_S = 4
_NEG = -1e30


def _silu(x):
    return x * jax.nn.sigmoid(x)


def _dot_tt(a, b):
    return jax.lax.dot_general(a, b, (((0,), (0,)), ((), ())),
                               preferred_element_type=jnp.float32)


def _dot_nt(a, b):
    return jax.lax.dot_general(a, b, (((1,), (1,)), ((), ())),
                               preferred_element_type=jnp.float32)


def _block_mean(x, bs):
    n, l = x.shape
    return jnp.mean(x.reshape(n // bs, bs, l), axis=1)


def _head_forward(q16, k16, v16, kc, kc0, vc0, causal_t):
    N, D = q16.shape
    nb = N // _BS
    scale = D ** (-0.5)
    f32 = jnp.float32
    bf16 = jnp.bfloat16

    s_sel_t = _dot_nt(kc.astype(bf16), q16) * scale
    j_sub = jax.lax.broadcasted_iota(jnp.int32, (nb, N), 0)
    m_lane = jax.lax.broadcasted_iota(jnp.int32, (nb, N), 1)
    blk_causal_t = (m_lane // _BS) >= j_sub
    s_m = jnp.where(blk_causal_t, s_sel_t, _NEG)
    sel_t = jnp.zeros((nb, N), dtype=jnp.bool_)
    for _ in range(_S):
        smax = jnp.max(s_m, axis=0, keepdims=True)
        is_max = s_m == smax
        first = jnp.min(jnp.where(is_max, j_sub, nb), axis=0, keepdims=True)
        pick = j_sub == first
        valid = smax > (_NEG * 0.5)
        sel_t = jnp.logical_or(sel_t, jnp.logical_and(pick, valid))
        s_m = jnp.where(pick, _NEG, s_m)

    sc0 = _dot_nt(q16, kc0.astype(bf16)) * scale
    qb = jax.lax.broadcasted_iota(jnp.int32, (N, nb), 0) // _BS
    jb = jax.lax.broadcasted_iota(jnp.int32, (N, nb), 1)
    p_cmp = jnp.where(qb >= jb, _silu(sc0), 0.0)
    o_cmp = jnp.dot(p_cmp.astype(bf16), vc0.astype(bf16),
                    preferred_element_type=f32)

    s_tok_t = (_dot_nt(k16, q16) * scale).astype(bf16)
    blk_of_t = jax.lax.broadcasted_iota(jnp.int32, (N, nb), 0) // _BS
    j_col = jax.lax.broadcasted_iota(jnp.int32, (N, nb), 1)
    ind_tok = (blk_of_t == j_col).astype(bf16)
    sel_exp_t = jnp.dot(ind_tok, sel_t.astype(bf16),
                        preferred_element_type=f32).astype(bf16)
    p_t = _silu(s_tok_t) * (sel_exp_t * causal_t)
    o_slc = _dot_tt(p_t, v16)

    return o_cmp, o_slc


def _fwd(q_ref, k_ref, v_ref, k0_ref, v0_ref, gw_ref, causal_ref, o_ref):
    f32 = jnp.float32
    bf16 = jnp.bfloat16
    _, N, H, D = q_ref.shape
    HD = H * D
    causal_t = causal_ref[...]

    qf = q_ref[0].reshape(N, HD)
    kf = k_ref[0].reshape(N, HD)
    vf = v_ref[0].reshape(N, HD)
    k0f = k0_ref[0].reshape(N, HD)
    v0f = v0_ref[0].reshape(N, HD)
    qf16 = qf.astype(bf16)
    kf16 = kf.astype(bf16)
    vf16 = vf.astype(bf16)

    kcf = _block_mean(kf, _BS)
    kc0f = _block_mean(k0f, _BS)
    vc0f = _block_mean(v0f, _BS)

    gates = jax.nn.sigmoid(jnp.dot(qf16, gw_ref[...],
                                   preferred_element_type=f32))

    l_row = jax.lax.broadcasted_iota(jnp.int32, (HD, D), 0)
    d_col = jax.lax.broadcasted_iota(jnp.int32, (HD, D), 1)
    for h in range(H):
        e_h = (l_row == d_col + h * D).astype(bf16)
        q16 = jnp.dot(qf16, e_h, preferred_element_type=f32).astype(bf16)
        k16 = jnp.dot(kf16, e_h, preferred_element_type=f32).astype(bf16)
        v16 = jnp.dot(vf16, e_h, preferred_element_type=f32).astype(bf16)
        kc = kcf[:, h * D:(h + 1) * D]
        kc0 = kc0f[:, h * D:(h + 1) * D]
        vc0 = vc0f[:, h * D:(h + 1) * D]
        o_cmp, o_slc = _head_forward(q16, k16, v16, kc, kc0, vc0, causal_t)
        g_cmp = gates[:, 3 * h:3 * h + 1]
        g_slc = gates[:, 3 * h + 1:3 * h + 2]
        o_ref[0, :, h, :] = o_cmp * g_cmp + o_slc * g_slc


def kernel(jagged_q, jagged_k, jagged_v, jagged_u, padded_q, padded_k,
           padded_v, x_offsets, gate_w, padding_mask):
    B, N, H, D = padded_q.shape
    causal_t = (jnp.arange(N)[None, :] >= jnp.arange(N)[:, None]
                ).astype(jnp.bfloat16)
    gw_big = jax.scipy.linalg.block_diag(
        *[gate_w[h] for h in range(H)]).astype(jnp.bfloat16)

    bspec = pl.BlockSpec((1, N, H, D), lambda b: (b, 0, 0, 0))
    b0spec = pl.BlockSpec((1, N, H, D), lambda b: (0, 0, 0, 0))
    gwspec = pl.BlockSpec((H * D, 3 * H), lambda b: (0, 0))
    cspec = pl.BlockSpec((N, N), lambda b: (0, 0))

    out = pl.pallas_call(
        _fwd,
        grid=(B,),
        in_specs=[bspec, bspec, bspec, b0spec, b0spec, gwspec, cspec],
        out_specs=bspec,
        out_shape=jax.ShapeDtypeStruct((B, N, H, D), jnp.float32),
        compiler_params=pltpu.CompilerParams(
            dimension_semantics=("parallel",)),
    )(padded_q, padded_k, padded_v, padded_k, padded_v, gw_big, causal_t)

    return out.reshape(B * N, H, D)


# speedup vs baseline: 1.1619x; 1.1619x over previous
import jax
import jax.numpy as jnp
from jax.experimental import pallas as pl
from jax.experimental.pallas import tpu as pltpu

_BS = 32
_S = 4
_NEG = -1e30


def _silu(x):
    return x * jax.nn.sigmoid(x)


def _dot_tt(a, b):
    return jax.lax.dot_general(a, b, (((0,), (0,)), ((), ())),
                               preferred_element_type=jnp.float32)


def _dot_nt(a, b):
    return jax.lax.dot_general(a, b, (((1,), (1,)), ((), ())),
                               preferred_element_type=jnp.float32)


def _fwd(q_ref, k_ref, v_ref, k0_ref, v0_ref, gw_ref, causal_ref, o_ref):
    q = q_ref[0, 0]
    k = k_ref[0, 0]
    v = v_ref[0, 0]
    k0 = k0_ref[0, 0]
    v0 = v0_ref[0, 0]
    gw = gw_ref[0]
    causal_t = causal_ref[...]

    N, D = q.shape
    nb = N // _BS
    scale = D ** (-0.5)
    f32 = jnp.float32
    bf16 = jnp.bfloat16

    kc = jnp.mean(k.reshape(nb, _BS, D), axis=1)
    kc0 = jnp.mean(k0.reshape(nb, _BS, D), axis=1)
    vc0 = jnp.mean(v0.reshape(nb, _BS, D), axis=1)

    q16 = q.astype(bf16)
    k16 = k.astype(bf16)
    v16 = v.astype(bf16)

    gates = jax.nn.sigmoid(jnp.dot(q16, gw.astype(bf16),
                                   preferred_element_type=f32))
    g_cmp = gates[:, 0:1]
    g_slc = gates[:, 1:2]

    s_sel_t = _dot_nt(kc.astype(bf16), q16) * scale
    j_sub = jax.lax.broadcasted_iota(jnp.int32, (nb, N), 0)
    m_lane = jax.lax.broadcasted_iota(jnp.int32, (nb, N), 1)
    blk_causal_t = (m_lane // _BS) >= j_sub
    s_m = jnp.where(blk_causal_t, s_sel_t, _NEG)
    sel_t = jnp.zeros((nb, N), dtype=jnp.bool_)
    for _ in range(_S):
        smax = jnp.max(s_m, axis=0, keepdims=True)
        is_max = s_m == smax
        first = jnp.min(jnp.where(is_max, j_sub, nb), axis=0, keepdims=True)
        pick = j_sub == first
        valid = smax > (_NEG * 0.5)
        sel_t = jnp.logical_or(sel_t, jnp.logical_and(pick, valid))
        s_m = jnp.where(pick, _NEG, s_m)

    sc0 = _dot_nt(q16, kc0.astype(bf16)) * scale
    qb = jax.lax.broadcasted_iota(jnp.int32, (N, nb), 0) // _BS
    jb = jax.lax.broadcasted_iota(jnp.int32, (N, nb), 1)
    p_cmp = jnp.where(qb >= jb, _silu(sc0), 0.0)
    o_cmp = jnp.dot(p_cmp.astype(bf16), vc0.astype(bf16),
                    preferred_element_type=f32)

    s_tok_t = (_dot_nt(k16, q16) * scale).astype(bf16)
    blk_of_t = jax.lax.broadcasted_iota(jnp.int32, (N, nb), 0) // _BS
    j_col = jax.lax.broadcasted_iota(jnp.int32, (N, nb), 1)
    ind_tok = (blk_of_t == j_col).astype(bf16)
    sel_exp_t = jnp.dot(ind_tok, sel_t.astype(bf16),
                        preferred_element_type=f32).astype(bf16)
    p_t = _silu(s_tok_t) * (sel_exp_t * causal_t)
    o_slc = _dot_tt(p_t, v16)

    o_ref[0, :, 0, 0, :] = o_cmp * g_cmp + o_slc * g_slc


def kernel(jagged_q, jagged_k, jagged_v, jagged_u, padded_q, padded_k,
           padded_v, x_offsets, gate_w, padding_mask):
    B, N, H, D = padded_q.shape
    qt = padded_q.transpose(0, 2, 1, 3)
    kt = padded_k.transpose(0, 2, 1, 3)
    vt = padded_v.transpose(0, 2, 1, 3)
    causal_t = (jnp.arange(N)[None, :] >= jnp.arange(N)[:, None]
                ).astype(jnp.bfloat16)

    bhspec = pl.BlockSpec((1, 1, N, D), lambda h, b: (b, h, 0, 0))
    b0spec = pl.BlockSpec((1, 1, N, D), lambda h, b: (0, h, 0, 0))
    gwspec = pl.BlockSpec((1, D, 3), lambda h, b: (h, 0, 0))
    cspec = pl.BlockSpec((N, N), lambda h, b: (0, 0))
    ospec = pl.BlockSpec((1, N, 1, 1, D), lambda h, b: (b, 0, h, 0, 0))

    out = pl.pallas_call(
        _fwd,
        grid=(H, B),
        in_specs=[bhspec, bhspec, bhspec, b0spec, b0spec, gwspec, cspec],
        out_specs=ospec,
        out_shape=jax.ShapeDtypeStruct((B, N, H, 1, D), jnp.float32),
        compiler_params=pltpu.CompilerParams(
            dimension_semantics=("parallel", "parallel")),
    )(qt, kt, vt, kt, vt, gate_w, causal_t)

    return out.reshape(B * N, H, D)


# speedup vs baseline: 1.2377x; 1.0653x over previous
import jax
import jax.numpy as jnp
from jax.experimental import pallas as pl
from jax.experimental.pallas import tpu as pltpu

_BS = 32
_S = 4
_NEG = -1e30


def _silu(x):
    return x * jax.nn.sigmoid(x)


def _dot_tt(a, b):
    return jax.lax.dot_general(a, b, (((0,), (0,)), ((), ())),
                               preferred_element_type=jnp.float32)


def _dot_nt(a, b):
    return jax.lax.dot_general(a, b, (((1,), (1,)), ((), ())),
                               preferred_element_type=jnp.float32)


def _fwd(q_ref, k_ref, v_ref, k0_ref, v0_ref, gw_ref, causal_ref, o_ref):
    q16 = q_ref[0, 0]
    k = k_ref[0, 0]
    v16 = v_ref[0, 0]
    k0 = k0_ref[0, 0]
    v016 = v0_ref[0, 0]
    gw = gw_ref[0]
    causal_t = causal_ref[...]

    N, D = q16.shape
    nb = N // _BS
    scale = D ** (-0.5)
    f32 = jnp.float32
    bf16 = jnp.bfloat16

    kc = jnp.mean(k.reshape(nb, _BS, D), axis=1)
    kc0 = jnp.mean(k0.reshape(nb, _BS, D), axis=1)
    vc0 = jnp.mean(v016.astype(f32).reshape(nb, _BS, D), axis=1)

    k16 = k.astype(bf16)

    gates = jax.nn.sigmoid(jnp.dot(q16, gw.astype(bf16),
                                   preferred_element_type=f32))
    g_cmp = gates[:, 0:1]
    g_slc = gates[:, 1:2]

    s_sel_t = _dot_nt(kc.astype(bf16), q16) * scale
    j_sub = jax.lax.broadcasted_iota(jnp.int32, (nb, N), 0)
    m_lane = jax.lax.broadcasted_iota(jnp.int32, (nb, N), 1)
    blk_causal_t = (m_lane // _BS) >= j_sub
    s_m = jnp.where(blk_causal_t, s_sel_t, _NEG)
    sel_t = jnp.zeros((nb, N), dtype=jnp.bool_)
    for _ in range(_S):
        smax = jnp.max(s_m, axis=0, keepdims=True)
        is_max = s_m == smax
        first = jnp.min(jnp.where(is_max, j_sub, nb), axis=0, keepdims=True)
        pick = j_sub == first
        valid = smax > (_NEG * 0.5)
        sel_t = jnp.logical_or(sel_t, jnp.logical_and(pick, valid))
        s_m = jnp.where(pick, _NEG, s_m)

    sc0 = _dot_nt(q16, kc0.astype(bf16)) * scale
    qb = jax.lax.broadcasted_iota(jnp.int32, (N, nb), 0) // _BS
    jb = jax.lax.broadcasted_iota(jnp.int32, (N, nb), 1)
    p_cmp = jnp.where(qb >= jb, _silu(sc0), 0.0)
    o_cmp = jnp.dot(p_cmp.astype(bf16), vc0.astype(bf16),
                    preferred_element_type=f32)

    s_tok_t = (_dot_nt(k16, q16) * scale).astype(bf16)
    blk_of_t = jax.lax.broadcasted_iota(jnp.int32, (N, nb), 0) // _BS
    j_col = jax.lax.broadcasted_iota(jnp.int32, (N, nb), 1)
    ind_tok = (blk_of_t == j_col).astype(bf16)
    sel_exp_t = jnp.dot(ind_tok, sel_t.astype(bf16),
                        preferred_element_type=f32).astype(bf16)
    p_t = _silu(s_tok_t) * (sel_exp_t * causal_t)
    o_slc = _dot_tt(p_t, v16)

    o_ref[0, :, 0, 0, :] = o_cmp * g_cmp + o_slc * g_slc


def kernel(jagged_q, jagged_k, jagged_v, jagged_u, padded_q, padded_k,
           padded_v, x_offsets, gate_w, padding_mask):
    B, N, H, D = padded_q.shape
    qt = padded_q.transpose(0, 2, 1, 3).astype(jnp.bfloat16)
    kt = padded_k.transpose(0, 2, 1, 3)
    vt = padded_v.transpose(0, 2, 1, 3).astype(jnp.bfloat16)
    causal_t = (jnp.arange(N)[None, :] >= jnp.arange(N)[:, None]
                ).astype(jnp.bfloat16)

    bhspec = pl.BlockSpec((1, 1, N, D), lambda h, b: (b, h, 0, 0))
    b0spec = pl.BlockSpec((1, 1, N, D), lambda h, b: (0, h, 0, 0))
    gwspec = pl.BlockSpec((1, D, 3), lambda h, b: (h, 0, 0))
    cspec = pl.BlockSpec((N, N), lambda h, b: (0, 0))
    ospec = pl.BlockSpec((1, N, 1, 1, D), lambda h, b: (b, 0, h, 0, 0))

    out = pl.pallas_call(
        _fwd,
        grid=(H, B),
        in_specs=[bhspec, bhspec, bhspec, b0spec, b0spec, gwspec, cspec],
        out_specs=ospec,
        out_shape=jax.ShapeDtypeStruct((B, N, H, 1, D), jnp.float32),
        compiler_params=pltpu.CompilerParams(
            dimension_semantics=("parallel", "parallel")),
    )(qt, kt, vt, kt, vt, gate_w, causal_t)

    return out.reshape(B * N, H, D)
